# Initial kernel scaffold; baseline (speedup 1.0000x reference)
import jax
import jax.numpy as jnp
from jax.experimental import pallas as pl


def kernel(x, edge_index, edge_weight, W1, b1, W2, b2, W3, b3, g1, be1, g2, be2, g3, be3, Wc, bc):
    raise NotImplementedError("write your pallas kernel here")


# speedup vs baseline: 6.5945x; 6.5945x over previous
import functools

import jax
import jax.numpy as jnp
from jax import lax
from jax.experimental import pallas as pl
from jax.experimental.pallas import tpu as pltpu
from jax.experimental.pallas import tpu_sc as plsc

_N = 10000
_E = 320000
_D = 128
_H = 128
_OUT = 64
_EPS = 1e-5

_NC = 2
_NS = 16
_NW = _NC * _NS
_NP = 10240
_RPT = _NP // _NS
_EPW = _E // _NW
_CH = 80
_NCHUNK = _EPW // _CH

_GD = lax.GatherDimensionNumbers(offset_dims=(), collapsed_slice_dims=(0,),
                                 start_index_map=(0,))


def _lane_bcast(vec16, j):
    idx = jnp.full((16, 1), j, jnp.int32)
    return lax.gather(vec16, idx, _GD, slice_sizes=(1,),
                      mode=lax.GatherScatterMode.PROMISE_IN_BOUNDS)


def _mk_mesh():
    return plsc.VectorSubcoreMesh(core_axis_name="c", subcore_axis_name="s",
                                  num_cores=_NC, num_subcores=_NS)


def _deg_sc(dst, ew):
    @functools.partial(
        pl.kernel,
        out_type=jax.ShapeDtypeStruct((_NC, _NP, 128), jnp.float32),
        mesh=_mk_mesh(),
        scratch_types=[
            pltpu.VMEM((_CH,), jnp.int32),
            pltpu.VMEM((_CH,), jnp.float32),
            pltpu.VMEM((_CH, 128), jnp.float32),
            pltpu.VMEM_SHARED((_NP, 128), jnp.float32),
        ],
    )
    def k(dst_hbm, ew_hbm, out_hbm, dst_v, ew_v, rows_v, acc_sh):
        cid = lax.axis_index("c")
        sid = lax.axis_index("s")
        wid = sid * _NC + cid
        zeros16 = jnp.zeros((16,), jnp.float32)

        @pl.loop(0, _CH)
        def _zero(r):
            for f in range(8):
                rows_v[r, pl.ds(16 * f, 16)] = zeros16

        base_r = sid * _RPT
        for b in range(_RPT // _CH):
            pltpu.sync_copy(rows_v, acc_sh.at[pl.ds(base_r + b * _CH, _CH)])
        plsc.subcore_barrier()

        ebase = wid * _EPW

        @pl.loop(0, _NCHUNK)
        def _chunk(t):
            off = ebase + t * _CH
            pltpu.sync_copy(dst_hbm.at[pl.ds(off, _CH)], dst_v)
            pltpu.sync_copy(ew_hbm.at[pl.ds(off, _CH)], ew_v)

            for b in range(_CH // 16):
                ew16 = ew_v[pl.ds(16 * b, 16)]
                for j in range(16):
                    sl0 = pl.ds(0, 16)
                    rows_v[16 * b + j, sl0] = (rows_v[16 * b + j, sl0] * 0.0
                                               + _lane_bcast(ew16, j))

            pltpu.sync_copy(rows_v, acc_sh.at[dst_v], add=True)

        plsc.subcore_barrier()
        pltpu.sync_copy(acc_sh.at[pl.ds(base_r, _RPT)],
                        out_hbm.at[cid, pl.ds(base_r, _RPT)])

    return k(dst, ew)


def _spmm_sc(h, src, dst, ew):
    @functools.partial(
        pl.kernel,
        out_type=jax.ShapeDtypeStruct((_NC, _NP, _H), jnp.float32),
        mesh=_mk_mesh(),
        scratch_types=[
            pltpu.VMEM((_CH,), jnp.int32),
            pltpu.VMEM((_CH,), jnp.int32),
            pltpu.VMEM((_CH,), jnp.float32),
            pltpu.VMEM((_CH, _H), jnp.float32),
            pltpu.VMEM_SHARED((_NP, _H), jnp.float32),
            pltpu.SemaphoreType.DMA,
        ],
    )
    def k(h_hbm, src_hbm, dst_hbm, ew_hbm, out_hbm,
          src_v, dst_v, ew_v, rows_v, acc_sh, sem):
        cid = lax.axis_index("c")
        sid = lax.axis_index("s")
        wid = sid * _NC + cid
        zeros16 = jnp.zeros((16,), jnp.float32)

        @pl.loop(0, _CH)
        def _zero(r):
            for f in range(_H // 16):
                rows_v[r, pl.ds(16 * f, 16)] = zeros16

        base_r = sid * _RPT
        for b in range(_RPT // _CH):
            pltpu.sync_copy(rows_v, acc_sh.at[pl.ds(base_r + b * _CH, _CH)])
        plsc.subcore_barrier()

        ebase = wid * _EPW

        @pl.loop(0, _NCHUNK)
        def _chunk(t):
            off = ebase + t * _CH
            pltpu.sync_copy(src_hbm.at[pl.ds(off, _CH)], src_v)
            pltpu.sync_copy(dst_hbm.at[pl.ds(off, _CH)], dst_v)
            pltpu.sync_copy(ew_hbm.at[pl.ds(off, _CH)], ew_v)
            pltpu.async_copy(h_hbm.at[src_v], rows_v, sem).wait()

            for b in range(_CH // 16):
                ew16 = ew_v[pl.ds(16 * b, 16)]
                for j in range(16):
                    ewb = _lane_bcast(ew16, j)
                    for f in range(_H // 16):
                        sl = pl.ds(16 * f, 16)
                        rows_v[16 * b + j, sl] = rows_v[16 * b + j, sl] * ewb

            pltpu.sync_copy(rows_v, acc_sh.at[dst_v], add=True)

        plsc.subcore_barrier()
        pltpu.sync_copy(acc_sh.at[pl.ds(base_r, _RPT)],
                        out_hbm.at[cid, pl.ds(base_r, _RPT)])

    return k(h, src, dst, ew)


def _tc_first(x, W1, b1, deg):
    def body(x_ref, w_ref, b_ref, deg_ref, dinv_ref, h_ref):
        dsum = deg_ref[0, :_N, 0:1] + deg_ref[1, :_N, 0:1]
        dinv = jnp.where(dsum > 0, lax.rsqrt(dsum), 0.0)
        dinv_ref[...] = dinv
        h = jnp.dot(x_ref[...], w_ref[...],
                    preferred_element_type=jnp.float32) + b_ref[...]
        h_ref[...] = h * dinv

    return pl.pallas_call(
        body,
        out_shape=[
            jax.ShapeDtypeStruct((_N, 1), jnp.float32),
            jax.ShapeDtypeStruct((_N, _H), jnp.float32),
        ],
    )(x, W1, b1, deg)


def _tc_mid(acc, dinv, g, be, W, b):
    def body(acc_ref, dinv_ref, g_ref, be_ref, w_ref, b_ref, x_ref, h_ref):
        dinv = dinv_ref[...]
        y = (acc_ref[0, :_N, :] + acc_ref[1, :_N, :]) * dinv
        m = jnp.mean(y, axis=0, keepdims=True)
        c = y - m
        v = jnp.mean(c * c, axis=0, keepdims=True)
        xk = jnp.maximum(g_ref[...] * c * lax.rsqrt(v + _EPS) + be_ref[...],
                         0.0)
        x_ref[...] = xk
        h_ref[...] = (jnp.dot(xk, w_ref[...],
                              preferred_element_type=jnp.float32)
                      + b_ref[...]) * dinv

    return pl.pallas_call(
        body,
        out_shape=[
            jax.ShapeDtypeStruct((_N, _H), jnp.float32),
            jax.ShapeDtypeStruct((_N, _H), jnp.float32),
        ],
    )(acc, dinv, g, be, W, b)


def _tc_last(acc, dinv, g, be, x1, x2, Wc, bc):
    def body(acc_ref, dinv_ref, g_ref, be_ref, x1_ref, x2_ref,
             wc_ref, bc_ref, out_ref):
        y = (acc_ref[0, :_N, :] + acc_ref[1, :_N, :]) * dinv_ref[...]
        m = jnp.mean(y, axis=0, keepdims=True)
        c = y - m
        v = jnp.mean(c * c, axis=0, keepdims=True)
        x3 = jnp.maximum(g_ref[...] * c * lax.rsqrt(v + _EPS) + be_ref[...],
                         0.0)
        out = (jnp.dot(x1_ref[...], wc_ref[0:_H, :],
                       preferred_element_type=jnp.float32)
               + jnp.dot(x2_ref[...], wc_ref[_H:2 * _H, :],
                         preferred_element_type=jnp.float32)
               + jnp.dot(x3, wc_ref[2 * _H:3 * _H, :],
                         preferred_element_type=jnp.float32))
        out_ref[...] = out + bc_ref[...]

    return pl.pallas_call(
        body,
        out_shape=jax.ShapeDtypeStruct((_N, _OUT), jnp.float32),
    )(acc, dinv, g, be, x1, x2, Wc, bc)


def kernel(x, edge_index, edge_weight, W1, b1, W2, b2, W3, b3,
           g1, be1, g2, be2, g3, be3, Wc, bc):
    src = edge_index[0]
    dst = edge_index[1]

    deg = _deg_sc(dst, edge_weight)
    dinv, h1 = _tc_first(x, W1, b1, deg)
    acc1 = _spmm_sc(h1, src, dst, edge_weight)
    x1, h2 = _tc_mid(acc1, dinv, g1, be1, W2, b2)
    acc2 = _spmm_sc(h2, src, dst, edge_weight)
    x2, h3 = _tc_mid(acc2, dinv, g2, be2, W3, b3)
    acc3 = _spmm_sc(h3, src, dst, edge_weight)
    return _tc_last(acc3, dinv, g3, be3, x1, x2, Wc, bc)
